# baseline (speedup 1.0000x reference)
import jax
import jax.numpy as jnp
from jax import lax
from jax.experimental import pallas as pl
from jax.experimental.pallas import tpu as pltpu, tpu_sc as plsc

N = 4096
B = 4
TOTAL = N * (N - 1) // 2
NT = TOTAL // 128
NW = 32
SPAN_T = 2048
CHUNK_T = 64
CHUNK = CHUNK_T * 128
TAIL_T = 48
NCHUNK = SPAN_T // CHUNK_T
NFULL = NT // CHUNK_T
K_BAL = 1200


def _row_of_py(p):
    lo, hi = 0, N - 1
    for _ in range(12):
        mid = (lo + hi) >> 1
        if (mid * ((2 * N - 1) - mid)) >> 1 <= p:
            lo = mid
        else:
            hi = mid
    return lo


def _balanced_bounds():
    cost = [c * CHUNK + K_BAL * _row_of_py(c * CHUNK) for c in range(NFULL + 1)]
    total = cost[NFULL]
    bounds = [0]
    for w in range(1, NW):
        target = total * w / NW
        c = next(i for i in range(NFULL + 1) if cost[i] >= target)
        c = max(c, bounds[-1] + 2)
        c = min(c, NFULL - 2 * (NW - w))
        bounds.append(c)
    bounds.append(NFULL)
    return bounds


BOUNDS = _balanced_bounds()


def _off(i):
    return (i * ((2 * N - 1) - i)) >> 1


def _row_of(p):
    lo = jnp.int32(0)
    hi = jnp.int32(N - 1)
    for _ in range(12):
        mid = (lo + hi) >> 1
        le = _off(mid) <= p
        lo = jnp.where(le, mid, lo)
        hi = jnp.where(le, hi, mid)
    return lo


def _body(in_hbm, out_hbm, in_v, buf_v, dma_sem):
    wid = lax.axis_index("c") * 16 + lax.axis_index("s")
    pltpu.sync_copy(in_hbm, in_v.at[pl.ds(0, B * N)])
    lane = lax.iota(jnp.int32, 16)

    def scatter_part(slot, b, dpos0, ln, src0, ai):
        dposv = dpos0 + lane
        mask = lane < ln
        sv = jnp.full((16,), slot, jnp.int32)
        tv = dposv >> 7
        cv = dposv & 127
        bv = jnp.full((16,), b, jnp.int32)
        val = in_v[pl.ds(src0, 16)] - ai
        plsc.store_scatter(buf_v, [sv, tv, bv, cv], val, mask=mask)

    def fill(slot, p0, ntiles):
        p1 = p0 + ntiles * 128
        i0 = _row_of(p0)

        def row_cond(st):
            _i, off_i = st
            return off_i < p1

        def row_body(st):
            i, off_i = st
            off_next = off_i + (N - 1 - i)
            seg_start = jnp.maximum(off_i, p0)
            seg_end = jnp.minimum(off_next, p1)
            d0 = seg_start - p0
            dend = seg_end - p0
            ja = i + 1 + (seg_start - off_i)
            dh = jnp.minimum((d0 + 15) & ~15, dend)
            df = dh + ((dend - dh) & ~15)
            dt0 = jnp.minimum((dh + 127) & ~127, df)
            dt1 = dt0 + ((df - dt0) & ~127)
            npre = (dt0 - dh) >> 4
            ntile = (dt1 - dt0) >> 7
            npost = (df - dt1) >> 4
            ais = [in_v[pl.ds(b * N + i, 16)][0] for b in range(B)]
            srcs = [b * N + ja - d0 for b in range(B)]

            for b in range(B):
                scatter_part(slot, b, d0, dh - d0, srcs[b] + d0, ais[b])

            @plsc.parallel_loop(0, npre)
            def pre_body(k):
                d = dh + k * 16
                t = d >> 7
                c = d & 127
                for b in range(B):
                    buf_v[slot, t, b, pl.ds(c, 16)] = (
                        in_v[pl.ds(srcs[b] + d, 16)] - ais[b])

            t_base = dt0 >> 7

            @plsc.parallel_loop(0, ntile, unroll=4)
            def tile_body(k):
                t = t_base + k
                d = dt0 + k * 128
                for b in range(B):
                    sb = srcs[b] + d
                    for u in range(8):
                        buf_v[slot, t, b, pl.ds(u * 16, 16)] = (
                            in_v[pl.ds(sb + u * 16, 16)] - ais[b])

            @plsc.parallel_loop(0, npost)
            def post_body(k):
                d = dt1 + k * 16
                t = d >> 7
                c = d & 127
                for b in range(B):
                    buf_v[slot, t, b, pl.ds(c, 16)] = (
                        in_v[pl.ds(srcs[b] + d, 16)] - ais[b])

            for b in range(B):
                scatter_part(slot, b, df, dend - df, srcs[b] + df, ais[b])
            return i + 1, off_next

        lax.while_loop(row_cond, row_body, (i0, _off(i0)))

    def wait_drain(ntiles):
        pltpu.make_async_copy(
            buf_v.at[0, pl.ds(0, ntiles)],
            out_hbm.at[pl.ds(0, ntiles)],
            dma_sem,
        ).wait()

    c_begin = jnp.int32(BOUNDS[0])
    nchunk = jnp.int32(BOUNDS[1] - BOUNDS[0])
    for w in range(1, NW):
        c_begin = jnp.where(wid == w, BOUNDS[w], c_begin)
        nchunk = jnp.where(wid == w, BOUNDS[w + 1] - BOUNDS[w], nchunk)

    def chunk_body(m, _):
        t0 = (c_begin + m) * CHUNK_T
        p0 = t0 * 128
        slot = m & 1

        @pl.when(m >= 2)
        def _w():
            wait_drain(CHUNK_T)

        fill(slot, p0, CHUNK_T)
        pltpu.async_copy(
            buf_v.at[slot], out_hbm.at[pl.ds(t0, CHUNK_T)], dma_sem)
        return 0

    lax.fori_loop(0, nchunk, chunk_body, 0)

    @pl.when(wid == NW - 1)
    def _tail():
        wait_drain(CHUNK_T)
        slot = nchunk & 1
        t0 = NFULL * CHUNK_T
        fill(slot, t0 * 128, TAIL_T)
        pltpu.async_copy(
            buf_v.at[slot, pl.ds(0, TAIL_T)],
            out_hbm.at[pl.ds(t0, TAIL_T)],
            dma_sem,
        )

    wait_drain(CHUNK_T)

    @pl.when(wid < NW - 1)
    def _dfull():
        wait_drain(CHUNK_T)

    @pl.when(wid == NW - 1)
    def _dtail():
        wait_drain(TAIL_T)


@jax.jit
def kernel(inputs):
    mesh = plsc.VectorSubcoreMesh(core_axis_name="c", subcore_axis_name="s")
    f = pl.kernel(
        _body,
        out_type=jax.ShapeDtypeStruct((NT, B, 128), jnp.float32),
        mesh=mesh,
        compiler_params=pltpu.CompilerParams(needs_layout_passes=False),
        scratch_types=[
            pltpu.VMEM((B * N + 16,), jnp.float32),
            pltpu.VMEM((2, CHUNK_T, B, 128), jnp.float32),
            pltpu.SemaphoreType.DMA,
        ],
    )
    out3 = f(inputs.reshape(B * N))
    return out3.transpose(1, 0, 2).reshape(B, TOTAL)


# speedup vs baseline: 1.3998x; 1.3998x over previous
import jax
import jax.numpy as jnp
from jax import lax
from jax.experimental import pallas as pl
from jax.experimental.pallas import tpu as pltpu, tpu_sc as plsc

N = 4096
B = 4
TOTAL = N * (N - 1) // 2
NT = TOTAL // 128
NW = 32
SPAN_T = 2048
CHUNK_T = 64
CHUNK = CHUNK_T * 128
TAIL_T = 48
NCHUNK = SPAN_T // CHUNK_T
NFULL = NT // CHUNK_T
K_BAL = 1200


def _row_of_py(p):
    lo, hi = 0, N - 1
    for _ in range(12):
        mid = (lo + hi) >> 1
        if (mid * ((2 * N - 1) - mid)) >> 1 <= p:
            lo = mid
        else:
            hi = mid
    return lo


def _balanced_bounds():
    cost = [c * CHUNK + K_BAL * _row_of_py(c * CHUNK) for c in range(NFULL + 1)]
    total = cost[NFULL]
    bounds = [0]
    for w in range(1, NW):
        target = total * w / NW
        c = next(i for i in range(NFULL + 1) if cost[i] >= target)
        c = max(c, bounds[-1] + 2)
        c = min(c, NFULL - 2 * (NW - w))
        bounds.append(c)
    bounds.append(NFULL)
    return bounds


BOUNDS = _balanced_bounds()


def _off(i):
    return (i * ((2 * N - 1) - i)) >> 1


def _row_of(p):
    lo = jnp.int32(0)
    hi = jnp.int32(N - 1)
    for _ in range(12):
        mid = (lo + hi) >> 1
        le = _off(mid) <= p
        lo = jnp.where(le, mid, lo)
        hi = jnp.where(le, hi, mid)
    return lo


def _body(in_hbm, out_hbm, in_v, buf_v, dma_sem):
    wid = lax.axis_index("c") * 16 + lax.axis_index("s")
    pltpu.sync_copy(in_hbm, in_v.at[pl.ds(0, B * N)])
    lane = lax.iota(jnp.int32, 16)

    def scatter_part(slot, b, dpos0, ln, src0, ai):
        dposv = dpos0 + lane
        mask = lane < ln
        sv = jnp.full((16,), slot, jnp.int32)
        tv = dposv >> 7
        cv = dposv & 127
        bv = jnp.full((16,), b, jnp.int32)
        val = in_v[pl.ds(src0, 16)] - ai
        plsc.store_scatter(buf_v, [sv, tv, bv, cv], val, mask=mask)

    def fill(slot, p0, ntiles):
        p1 = p0 + ntiles * 128
        i0 = _row_of(p0)

        def row_cond(st):
            _i, off_i = st
            return off_i < p1

        def row_body(st):
            i, off_i = st
            off_next = off_i + (N - 1 - i)
            seg_start = jnp.maximum(off_i, p0)
            seg_end = jnp.minimum(off_next, p1)
            d0 = seg_start - p0
            dend = seg_end - p0
            ja = i + 1 + (seg_start - off_i)
            dh = jnp.minimum((d0 + 15) & ~15, dend)
            df = dh + ((dend - dh) & ~15)
            dt0 = jnp.minimum((dh + 127) & ~127, df)
            dt1 = dt0 + ((df - dt0) & ~127)
            npre = (dt0 - dh) >> 4
            ntile = (dt1 - dt0) >> 7
            npost = (df - dt1) >> 4
            ais = [in_v[pl.ds(b * N + i, 16)][0] for b in range(B)]
            srcs = [b * N + ja - d0 for b in range(B)]

            for b in range(B):
                scatter_part(slot, b, d0, dh - d0, srcs[b] + d0, ais[b])

            @plsc.parallel_loop(0, npre)
            def pre_body(k):
                d = dh + k * 16
                t = d >> 7
                c = d & 127
                for b in range(B):
                    buf_v[slot, t, b, pl.ds(c, 16)] = (
                        in_v[pl.ds(srcs[b] + d, 16)] - ais[b])

            t_base = dt0 >> 7

            @plsc.parallel_loop(0, ntile, unroll=2)
            def tile_body(k):
                t = t_base + k
                d = dt0 + k * 128
                for b in range(B):
                    sb = srcs[b] + d
                    for u in range(8):
                        buf_v[slot, t, b, pl.ds(u * 16, 16)] = (
                            in_v[pl.ds(sb + u * 16, 16)] - ais[b])

            @plsc.parallel_loop(0, npost)
            def post_body(k):
                d = dt1 + k * 16
                t = d >> 7
                c = d & 127
                for b in range(B):
                    buf_v[slot, t, b, pl.ds(c, 16)] = (
                        in_v[pl.ds(srcs[b] + d, 16)] - ais[b])

            for b in range(B):
                scatter_part(slot, b, df, dend - df, srcs[b] + df, ais[b])
            return i + 1, off_next

        lax.while_loop(row_cond, row_body, (i0, _off(i0)))

    def wait_drain(ntiles):
        pltpu.make_async_copy(
            buf_v.at[0, pl.ds(0, ntiles)],
            out_hbm.at[pl.ds(0, ntiles)],
            dma_sem,
        ).wait()

    c_begin = jnp.int32(BOUNDS[0])
    nchunk = jnp.int32(BOUNDS[1] - BOUNDS[0])
    for w in range(1, NW):
        c_begin = jnp.where(wid == w, BOUNDS[w], c_begin)
        nchunk = jnp.where(wid == w, BOUNDS[w + 1] - BOUNDS[w], nchunk)

    def chunk_body(m, _):
        t0 = (c_begin + m) * CHUNK_T
        p0 = t0 * 128
        slot = m & 1

        @pl.when(m >= 2)
        def _w():
            wait_drain(CHUNK_T)

        fill(slot, p0, CHUNK_T)
        pltpu.async_copy(
            buf_v.at[slot], out_hbm.at[pl.ds(t0, CHUNK_T)], dma_sem)
        return 0

    lax.fori_loop(0, nchunk, chunk_body, 0)

    @pl.when(wid == NW - 1)
    def _tail():
        wait_drain(CHUNK_T)
        slot = nchunk & 1
        t0 = NFULL * CHUNK_T
        fill(slot, t0 * 128, TAIL_T)
        pltpu.async_copy(
            buf_v.at[slot, pl.ds(0, TAIL_T)],
            out_hbm.at[pl.ds(t0, TAIL_T)],
            dma_sem,
        )

    wait_drain(CHUNK_T)

    @pl.when(wid < NW - 1)
    def _dfull():
        wait_drain(CHUNK_T)

    @pl.when(wid == NW - 1)
    def _dtail():
        wait_drain(TAIL_T)


@jax.jit
def kernel(inputs):
    mesh = plsc.VectorSubcoreMesh(core_axis_name="c", subcore_axis_name="s")
    f = pl.kernel(
        _body,
        out_type=jax.ShapeDtypeStruct((NT, B, 128), jnp.float32),
        mesh=mesh,
        compiler_params=pltpu.CompilerParams(needs_layout_passes=False),
        scratch_types=[
            pltpu.VMEM((B * N + 16,), jnp.float32),
            pltpu.VMEM((2, CHUNK_T, B, 128), jnp.float32),
            pltpu.SemaphoreType.DMA,
        ],
    )
    out3 = f(inputs.reshape(B * N))
    return out3.transpose(1, 0, 2).reshape(B, TOTAL)
